# Initial kernel scaffold; baseline (speedup 1.0000x reference)
import jax
import jax.numpy as jnp
from jax.experimental import pallas as pl


def kernel(x, edge_index, labels, times, W, b):
    raise NotImplementedError("write your pallas kernel here")


# speedup vs baseline: 4.5531x; 4.5531x over previous
import functools

import jax
import jax.numpy as jnp
from jax import lax
from jax.experimental import pallas as pl
from jax.experimental.pallas import tpu as pltpu
from jax.experimental.pallas import tpu_sc as plsc

N = 10000
E = 320000
D = 128
OUT = 128
T = 10
L = 40
SPLIT = 7
TL = T * L
TRSEG = SPLIT * L

NP = 10240
NC = 2
NS = 16
NW = NC * NS
RPT = NP // NS
B = 80
EPC = E // NC
EPS = EPC // NS
NBLK_E = EPS // B
TLP = 512
SRT = TLP // NS
RPW = NP // NW
NBLK_N = RPW // B

_mesh = plsc.VectorSubcoreMesh(core_axis_name="c", subcore_axis_name="s")


def _wid(c, s):
    return c * NS + s


@functools.partial(
    pl.kernel,
    out_type=jax.ShapeDtypeStruct((NW * NP,), jnp.float32),
    mesh=_mesh,
    scratch_types=[
        pltpu.VMEM((B,), jnp.int32),
        pltpu.VMEM((NP,), jnp.float32),
    ],
    compiler_params=pltpu.CompilerParams(needs_layout_passes=False),
)
def _deg_kernel(dst_hbm, out_hbm, dst_v, hist):
    c = lax.axis_index("c")
    s = lax.axis_index("s")
    zeros16 = jnp.zeros((16,), jnp.float32)

    def zbody(i, _):
        hist[pl.ds(i * 16, 16)] = zeros16
        return 0

    lax.fori_loop(0, NP // 16, zbody, 0)
    base0 = c * EPC + s * EPS
    ones16 = jnp.ones((16,), jnp.float32)

    def body(i, _):
        pltpu.sync_copy(dst_hbm.at[pl.ds(base0 + i * B, B)], dst_v)
        for j in range(B // 16):
            idx = dst_v[pl.ds(j * 16, 16)]
            plsc.addupdate_scatter(hist, [idx], ones16)
        return 0

    lax.fori_loop(0, NBLK_E, body, 0)
    pltpu.sync_copy(hist, out_hbm.at[pl.ds(_wid(c, s) * NP, NP)])


@functools.partial(
    pl.kernel,
    out_type=jax.ShapeDtypeStruct((NC * NP, D), jnp.float32),
    mesh=_mesh,
    scratch_types=[
        pltpu.VMEM((B,), jnp.int32),
        pltpu.VMEM((B,), jnp.int32),
        pltpu.VMEM((B, D), jnp.float32),
        pltpu.VMEM_SHARED((NP, D), jnp.float32),
        pltpu.SemaphoreType.DMA,
    ],
)
def _hop_kernel(g_hbm, src_hbm, dst_hbm, z_hbm, out_hbm,
                src_v, dst_v, rows_v, acc, sem):
    c = lax.axis_index("c")
    s = lax.axis_index("s")
    pltpu.sync_copy(z_hbm, acc.at[pl.ds(s * RPT, RPT)])
    plsc.subcore_barrier()
    base0 = c * EPC + s * EPS

    def body(i, _):
        base = base0 + i * B
        pltpu.sync_copy(src_hbm.at[pl.ds(base, B)], src_v)
        pltpu.sync_copy(dst_hbm.at[pl.ds(base, B)], dst_v)
        pltpu.async_copy(g_hbm.at[src_v], rows_v, sem).wait()
        pltpu.sync_copy(rows_v, acc.at[dst_v], add=True)
        return 0

    lax.fori_loop(0, NBLK_E, body, 0)
    plsc.subcore_barrier()
    pltpu.sync_copy(acc.at[pl.ds(s * RPT, RPT)],
                    out_hbm.at[pl.ds(c * NP + s * RPT, RPT)])


@functools.partial(
    pl.kernel,
    out_type=(
        jax.ShapeDtypeStruct((NC * TLP, D), jnp.float32),
        jax.ShapeDtypeStruct((NC * TLP, D), jnp.float32),
        jax.ShapeDtypeStruct((NC * TLP, D), jnp.float32),
    ),
    mesh=_mesh,
    scratch_types=[
        pltpu.VMEM((B,), jnp.int32),
        pltpu.VMEM((B, D), jnp.float32),
        pltpu.VMEM((B, D), jnp.float32),
        pltpu.VMEM((B, D), jnp.float32),
        pltpu.VMEM_SHARED((TLP, D), jnp.float32),
        pltpu.VMEM_SHARED((TLP, D), jnp.float32),
        pltpu.VMEM_SHARED((TLP, D), jnp.float32),
    ],
)
def _stats_kernel(h_hbm, q_hbm, ones_hbm, seg_hbm, z_hbm,
                  outh_hbm, outq_hbm, outc_hbm,
                  seg_v, rows_h, rows_q, ones_v, acc_h, acc_q, acc_c):
    c = lax.axis_index("c")
    s = lax.axis_index("s")
    pltpu.sync_copy(z_hbm, acc_h.at[pl.ds(s * SRT, SRT)])
    pltpu.sync_copy(z_hbm, acc_q.at[pl.ds(s * SRT, SRT)])
    pltpu.sync_copy(z_hbm, acc_c.at[pl.ds(s * SRT, SRT)])
    pltpu.sync_copy(ones_hbm, ones_v)
    plsc.subcore_barrier()
    base0 = _wid(c, s) * RPW

    def body(i, _):
        base = base0 + i * B
        pltpu.sync_copy(seg_hbm.at[pl.ds(base, B)], seg_v)
        pltpu.sync_copy(h_hbm.at[pl.ds(base, B)], rows_h)
        pltpu.sync_copy(q_hbm.at[pl.ds(base, B)], rows_q)
        pltpu.sync_copy(rows_h, acc_h.at[seg_v], add=True)
        pltpu.sync_copy(rows_q, acc_q.at[seg_v], add=True)
        pltpu.sync_copy(ones_v, acc_c.at[seg_v], add=True)
        return 0

    lax.fori_loop(0, NBLK_N, body, 0)
    plsc.subcore_barrier()
    sl_s = pl.ds(s * SRT, SRT)
    sl_o = pl.ds(c * TLP + s * SRT, SRT)
    pltpu.sync_copy(acc_h.at[sl_s], outh_hbm.at[sl_o])
    pltpu.sync_copy(acc_q.at[sl_s], outq_hbm.at[sl_o])
    pltpu.sync_copy(acc_c.at[sl_s], outc_hbm.at[sl_o])


@functools.partial(
    pl.kernel,
    out_type=jax.ShapeDtypeStruct((NP, D), jnp.float32),
    mesh=_mesh,
    scratch_types=[
        pltpu.VMEM((B,), jnp.int32),
        pltpu.VMEM((B, D), jnp.float32),
        pltpu.SemaphoreType.DMA,
    ],
)
def _gather_kernel(tbl_hbm, seg_hbm, out_hbm, seg_v, rows_v, sem):
    c = lax.axis_index("c")
    s = lax.axis_index("s")
    base0 = _wid(c, s) * RPW

    def body(i, _):
        base = base0 + i * B
        pltpu.sync_copy(seg_hbm.at[pl.ds(base, B)], seg_v)
        pltpu.async_copy(tbl_hbm.at[seg_v], rows_v, sem).wait()
        pltpu.sync_copy(rows_v, out_hbm.at[pl.ds(base, B)])
        return 0

    lax.fori_loop(0, NBLK_N, body, 0)


_BR = 256


def _final_body(h_ref, tmg_ref, af_ref, v_ref, bt_ref, out_ref):
    af = af_ref[...]
    hf = af * h_ref[...] + (1.0 - af) * tmg_ref[...]
    out_ref[...] = jax.lax.dot_general(
        hf, v_ref[...], (((1,), (0,)), ((), ())),
        preferred_element_type=jnp.float32,
        precision=jax.lax.Precision.HIGHEST,
    ) + bt_ref[...]


def _final_tc(h, tmg, af, V, btil):
    grid = (NP // _BR,)
    return pl.pallas_call(
        _final_body,
        grid=grid,
        in_specs=[
            pl.BlockSpec((_BR, D), lambda i: (i, 0)),
            pl.BlockSpec((_BR, D), lambda i: (i, 0)),
            pl.BlockSpec((_BR, 1), lambda i: (i, 0)),
            pl.BlockSpec((D, OUT), lambda i: (0, 0)),
            pl.BlockSpec((1, OUT), lambda i: (0, 0)),
        ],
        out_specs=pl.BlockSpec((_BR, OUT), lambda i: (i, 0)),
        out_shape=jax.ShapeDtypeStruct((NP, OUT), jnp.float32),
    )(h, tmg, af, V, btil)


def kernel(x, edge_index, labels, times, W, b):
    f32 = jnp.float32
    src = edge_index[0]
    dst = edge_index[1]
    zrow = jnp.zeros((RPT, D), f32)
    zst = jnp.zeros((SRT, D), f32)
    ones_bd = jnp.ones((B, D), f32)

    degp = _deg_kernel(dst)
    deg = jnp.sum(degp.reshape(NW, NP), axis=0)[:, None]
    norm = jnp.power(jnp.maximum(deg, 1.0), -0.5)

    xpad = jnp.zeros((NP, D), f32).at[:N].set(x)
    g = xpad * norm
    p = _hop_kernel(g, src, dst, zrow)
    s1 = p[:NP] + p[NP:]
    g = s1 * (norm * norm)
    p = _hop_kernel(g, src, dst, zrow)
    h = (p[:NP] + p[NP:]) * norm

    seg = times * L + labels
    segp = jnp.full((NP,), TLP - 1, jnp.int32).at[:N].set(seg)
    q = h * h
    oh, oq, oc = _stats_kernel(h, q, ones_bd, segp, zst)
    ssum = (oh[:TLP] + oh[TLP:])[:TL]
    ssq = (oq[:TLP] + oq[TLP:])[:TL]
    cnt = (oc[:TLP] + oc[TLP:])[:TL, 0]

    tr_cnt = cnt[:TRSEG].reshape(SPLIT, L)
    tr_sum = ssum[:TRSEG].reshape(SPLIT, L, D)
    tr_ssq = ssq[:TRSEG].reshape(SPLIT, L, D)
    test_cnt = jnp.sum(cnt[TRSEG:])
    test_sum = jnp.sum(ssum[TRSEG:], axis=0)
    test_ssq = jnp.sum(ssq[TRSEG:], axis=0)
    test_mean = test_sum / jnp.maximum(test_cnt, 1.0)
    test_var = (jnp.sum(test_ssq) - 2.0 * jnp.dot(test_mean, test_sum)
                + test_cnt * jnp.dot(test_mean, test_mean)
                ) / jnp.maximum(test_cnt - 1.0, 1.0)
    time_cnt = jnp.sum(tr_cnt, axis=1)
    ttm = jnp.sum(tr_sum, axis=1) / jnp.maximum(time_cnt, 1.0)[:, None]
    tm = tr_sum / jnp.maximum(tr_cnt, 1.0)[:, :, None]
    msq = jnp.sum(tr_cnt * jnp.sum((tm - ttm[:, None, :]) ** 2, axis=2), axis=1)
    rsq = jnp.sum(jnp.sum(tr_ssq, axis=2)
                  - 2.0 * jnp.sum(tm * tr_sum, axis=2)
                  + tr_cnt * jnp.sum(tm * tm, axis=2), axis=1)
    denom = jnp.maximum(time_cnt - 1.0, 1.0)
    alpha_sq = (test_var - msq / denom) / jnp.maximum(rsq / denom, 1e-6)
    alpha7 = jnp.where(alpha_sq > 0, jnp.sqrt(jnp.maximum(alpha_sq, 0.0)), 0.0)

    tot_cnt = jnp.sum(cnt)
    mu = (jnp.sum(tr_sum, axis=(0, 1)) + test_sum) / tot_cnt
    a2 = (alpha7 ** 2)[:, None, None]
    blend_ssq = a2 * tr_ssq + (1.0 - a2) * (tr_sum ** 2) \
        / jnp.maximum(tr_cnt, 1.0)[:, :, None]
    col_ssq = jnp.sum(blend_ssq, axis=(0, 1)) + test_ssq
    sigma = jnp.sqrt((col_ssq - tot_cnt * mu * mu) / (tot_cnt - 1.0))
    V = (W / sigma[None, :]).T
    btil = (b - (mu / sigma) @ W.T)[None, :]

    tm_tbl = jnp.zeros((TLP, D), f32).at[:TRSEG].set(tm.reshape(TRSEG, D))
    tmg = _gather_kernel(tm_tbl, segp)
    alpha10 = jnp.concatenate([alpha7, jnp.ones((T - SPLIT,), f32)])
    af = jnp.ones((NP, 1), f32).at[:N, 0].set(alpha10[times])

    out = _final_tc(h, tmg, af, V, btil)
    return out[:N]
